# baseline (speedup 1.0000x reference)
import functools

import jax
import jax.numpy as jnp
from jax.experimental import pallas as pl

N, A, C = 128, 8732, 81
SCALE_XY = 1.0 / 0.1
SCALE_WH = 1.0 / 0.2

AB = 512
J = (A + AB - 1) // AB


def _phase1_kernel(ploc_ref, plabel_ref, gloc_ref, glabel_ref, dboxes_ref,
                   con_neg_ref, stats_ref):
    j = pl.program_id(1)

    lbl = glabel_ref[0]
    lane = jax.lax.broadcasted_iota(jnp.int32, (1, AB), 1)
    valid = (j * AB + lane) < A
    posm = (lbl > 0) & valid

    x = plabel_ref[0]
    e = jnp.exp(x)
    s = jnp.sum(e, axis=0, keepdims=True)
    logz = jnp.log(s)
    cidx = jax.lax.broadcasted_iota(jnp.int32, (C, AB), 0)
    tl = jnp.sum(jnp.where(cidx == lbl, x, 0.0), axis=0, keepdims=True)
    con = logz - tl

    con_neg = jnp.where(posm | ~valid, 0.0, con)
    con_pos = jnp.where(posm, con, 0.0)

    p = ploc_ref[0]
    g = gloc_ref[0]
    db = dboxes_ref[0]
    gxy = SCALE_XY * (g[:2] - db[:2]) / db[2:]
    gwh = SCALE_WH * jnp.log(g[2:] / db[2:])
    vec = jnp.concatenate([gxy, gwh], axis=0)
    d = p - vec
    ad = jnp.abs(d)
    sl1 = jnp.sum(jnp.where(ad < 1.0, 0.5 * d * d, ad - 0.5),
                  axis=0, keepdims=True)
    ll = jnp.where(posm, sl1, 0.0)

    pos_f = jnp.where(posm, 1.0, 0.0)

    con_neg_ref[0] = con_neg

    @pl.when(j == 0)
    def _():
        stats_ref[...] = jnp.zeros_like(stats_ref)

    stats_ref[0, 0:1, :] += pos_f
    stats_ref[0, 1:2, :] += con_pos
    stats_ref[0, 2:3, :] += ll


def _phase2_kernel(con_neg_ref, stats_ref, out_ref):
    st = stats_ref[...]
    pos = jnp.sum(st[:, 0, :], axis=-1, keepdims=True)
    conm = jnp.sum(st[:, 1, :], axis=-1, keepdims=True)
    locm = jnp.sum(st[:, 2, :], axis=-1, keepdims=True)

    cn = con_neg_ref[...]
    ci = jax.lax.bitcast_convert_type(cn, jnp.int32)

    pos_i = pos.astype(jnp.int32)
    k = jnp.minimum(3 * pos_i, A)
    kk = jnp.maximum(k, 1).astype(jnp.float32)

    prefix = jnp.zeros((N, 1), jnp.int32)
    krem = kk
    for b in range(30, -1, -1):
        hi_mask = jnp.int32(-(1 << b))
        cand = prefix | jnp.int32(1 << b)
        cnt = jnp.sum(jnp.where((ci & hi_mask) == cand, 1.0, 0.0),
                      axis=1, keepdims=True)
        take = krem <= cnt
        prefix = jnp.where(take, cand, prefix)
        krem = jnp.where(take, krem, krem - cnt)

    v = jax.lax.bitcast_convert_type(prefix, jnp.float32)
    gt = cn > v
    t_cnt = jnp.sum(jnp.where(gt, 1.0, 0.0), axis=1, keepdims=True)
    ns = jnp.sum(jnp.where(gt, cn, 0.0), axis=1, keepdims=True)
    neg_total = ns + (k.astype(jnp.float32) - t_cnt) * v

    total = locm + conm + neg_total
    contrib = jnp.where(pos > 0, total / jnp.maximum(pos, 1e-6), 0.0)
    out_ref[...] = jnp.sum(contrib, keepdims=True).reshape(1, 1) / N


@jax.jit
def kernel(ploc, plabel, gloc, glabel, dboxes):
    glabel3 = glabel.reshape(N, 1, A)

    con_neg, stats = pl.pallas_call(
        _phase1_kernel,
        grid=(N, J),
        in_specs=[
            pl.BlockSpec((1, 4, AB), lambda n, j: (n, 0, j)),
            pl.BlockSpec((1, C, AB), lambda n, j: (n, 0, j)),
            pl.BlockSpec((1, 4, AB), lambda n, j: (n, 0, j)),
            pl.BlockSpec((1, 1, AB), lambda n, j: (n, 0, j)),
            pl.BlockSpec((1, 4, AB), lambda n, j: (0, 0, j)),
        ],
        out_specs=[
            pl.BlockSpec((1, 1, AB), lambda n, j: (n, 0, j)),
            pl.BlockSpec((1, 8, AB), lambda n, j: (n, 0, 0)),
        ],
        out_shape=[
            jax.ShapeDtypeStruct((N, 1, A), jnp.float32),
            jax.ShapeDtypeStruct((N, 8, AB), jnp.float32),
        ],
    )(ploc, plabel, gloc, glabel3, dboxes)

    out = pl.pallas_call(
        _phase2_kernel,
        out_shape=jax.ShapeDtypeStruct((1, 1), jnp.float32),
    )(con_neg.reshape(N, A), stats)
    return out[0, 0]


# speedup vs baseline: 2.6816x; 2.6816x over previous
import jax
import jax.numpy as jnp
from jax.experimental import pallas as pl

N, A, C = 128, 8732, 81
SCALE_XY = 1.0 / 0.1
SCALE_WH = 1.0 / 0.2

RW = 512
NFULL = A // RW
REM = A - NFULL * RW


def _phase1_kernel(ploc_ref, plabel_ref, gloc_ref, glabel_ref, dboxes_ref,
                   con_ref, locred_ref):
    lbl = glabel_ref[0]
    posm = lbl > 0

    x = plabel_ref[0]
    e = jnp.exp(x)
    s = jnp.sum(e, axis=0, keepdims=True)
    logz = jnp.log(s)
    cidx = jax.lax.broadcasted_iota(jnp.int32, (C, A), 0)
    tl = jnp.sum(jnp.where(cidx == lbl, x, 0.0), axis=0, keepdims=True)
    con_ref[0] = logz - tl

    p = ploc_ref[0]
    g = gloc_ref[0]
    db = dboxes_ref[0]
    gxy = SCALE_XY * (g[:2] - db[:2]) / db[2:]
    gwh = SCALE_WH * jnp.log(g[2:] / db[2:])
    vec = jnp.concatenate([gxy, gwh], axis=0)
    d = p - vec
    ad = jnp.abs(d)
    sl1 = jnp.sum(jnp.where(ad < 1.0, 0.5 * d * d, ad - 0.5),
                  axis=0, keepdims=True)
    ll = jnp.where(posm, sl1, 0.0)

    acc = jnp.zeros((1, RW), jnp.float32)
    for i in range(NFULL):
        acc = acc + ll[:, i * RW:(i + 1) * RW]
    rem = ll[:, NFULL * RW:]
    acc = acc + jnp.concatenate(
        [rem, jnp.zeros((1, RW - REM), jnp.float32)], axis=1)
    locred_ref[0] = acc


def _phase2_kernel(con_ref, glabel_ref, locred_ref, out_ref):
    lbl = glabel_ref[...]
    posm = lbl > 0
    con = con_ref[...]

    pos = jnp.sum(jnp.where(posm, 1.0, 0.0), axis=1, keepdims=True)
    conm = jnp.sum(jnp.where(posm, con, 0.0), axis=1, keepdims=True)
    locm = jnp.sum(locred_ref[...], axis=1, keepdims=True)

    cn = jnp.where(posm, 0.0, con)
    ci = jax.lax.bitcast_convert_type(cn, jnp.int32)

    pos_i = pos.astype(jnp.int32)
    k = jnp.minimum(3 * pos_i, A)
    kk = jnp.maximum(k, 1).astype(jnp.float32)

    prefix = jnp.zeros((N, 1), jnp.int32)
    krem = kk
    for b in range(30, -1, -1):
        hi_mask = jnp.int32(-(1 << b))
        cand = prefix | jnp.int32(1 << b)
        cnt = jnp.sum(jnp.where((ci & hi_mask) == cand, 1.0, 0.0),
                      axis=1, keepdims=True)
        take = krem <= cnt
        prefix = jnp.where(take, cand, prefix)
        krem = jnp.where(take, krem, krem - cnt)

    v = jax.lax.bitcast_convert_type(prefix, jnp.float32)
    gt = cn > v
    t_cnt = jnp.sum(jnp.where(gt, 1.0, 0.0), axis=1, keepdims=True)
    ns = jnp.sum(jnp.where(gt, cn, 0.0), axis=1, keepdims=True)
    neg_total = ns + (k.astype(jnp.float32) - t_cnt) * v

    total = locm + conm + neg_total
    contrib = jnp.where(pos > 0, total / jnp.maximum(pos, 1e-6), 0.0)
    out_ref[...] = jnp.sum(contrib, keepdims=True).reshape(1, 1) / N


@jax.jit
def kernel(ploc, plabel, gloc, glabel, dboxes):
    glabel3 = glabel.reshape(N, 1, A)

    con, locred = pl.pallas_call(
        _phase1_kernel,
        grid=(N,),
        in_specs=[
            pl.BlockSpec((1, 4, A), lambda n: (n, 0, 0)),
            pl.BlockSpec((1, C, A), lambda n: (n, 0, 0)),
            pl.BlockSpec((1, 4, A), lambda n: (n, 0, 0)),
            pl.BlockSpec((1, 1, A), lambda n: (n, 0, 0)),
            pl.BlockSpec((1, 4, A), lambda n: (0, 0, 0)),
        ],
        out_specs=[
            pl.BlockSpec((1, 1, A), lambda n: (n, 0, 0)),
            pl.BlockSpec((1, 1, RW), lambda n: (n, 0, 0)),
        ],
        out_shape=[
            jax.ShapeDtypeStruct((N, 1, A), jnp.float32),
            jax.ShapeDtypeStruct((N, 1, RW), jnp.float32),
        ],
    )(ploc, plabel, gloc, glabel3, dboxes)

    out = pl.pallas_call(
        _phase2_kernel,
        out_shape=jax.ShapeDtypeStruct((1, 1), jnp.float32),
    )(con.reshape(N, A), glabel, locred.reshape(N, RW))
    return out[0, 0]
